# baseline (speedup 1.0000x reference)
import functools

import jax
import jax.numpy as jnp
from jax.experimental import pallas as pl

_RS2 = 0.7071067811865476


def _g2(t):
    return t * (1.0 + jax.lax.erf(t))


def _ln(x, g, b, eps=1e-05):
    mu = jnp.mean(x, axis=-1, keepdims=True)
    var = jnp.mean((x - mu) ** 2, axis=-1, keepdims=True)
    return (x - mu) / jnp.sqrt(var + eps) * g + b


def _decoder_block(node_ref, edge_ref, mask_ref, a8_ref, mw0_ref, mb0_ref,
                   mw1_ref, mb1_ref, mw2_ref, mb2_ref, n1w_ref, n1b_ref,
                   dw0_ref, db0_ref, dw1_ref, db1_ref, n2w_ref, n2b_ref,
                   out_ref, *, num_layers):
    B, K, E = edge_ref.shape
    C = node_ref.shape[1]

    x = node_ref[...]
    e = edge_ref[...].reshape(B * K, E)
    msk = mask_ref[...]
    h = x
    te = e @ mw0_ref[0][:, 3 * C:].T
    for i in range(num_layers):
        w0 = mw0_ref[i]
        wh = w0[:, :C]
        wn = w0[:, C:2 * C]
        per_node = h @ wh.T + x @ wn.T + mb0_ref[i]
        t = te + jnp.broadcast_to(per_node[:, None, :],
                                  (B, K, C)).reshape(B * K, C)
        m = _g2(t)
        m = _g2(m @ mw1_ref[i].T + mb1_ref[i])
        m4 = m.reshape(B, K // 8, 8, C)
        xs = m4[:, 0]
        for j in range(1, K // 8):
            xs = xs + m4[:, j]
        s = a8_ref[...] @ xs.reshape(B * 8, C)
        if i + 1 < num_layers:
            te = e @ mw0_ref[i + 1][:, 3 * C:].T
        h = h + s @ mw2_ref[i].T + mb2_ref[i]
        h1 = _ln(h, n1w_ref[i], n1b_ref[i])
        dns = _g2(h1 @ dw0_ref[i].T + db0_ref[i])
        h = h1 + dns @ dw1_ref[i].T + db1_ref[i]
        h = msk * _ln(h, n2w_ref[i], n2b_ref[i])
    out_ref[...] = h


def kernel(node_features, edge_features, mask, m_w0, m_b0, m_w1, m_b1, m_w2,
           m_b2, n1_w, n1_b, d_w0, d_b0, d_w1, d_b1, n2_w, n2_b):
    N, K, E = edge_features.shape
    C = node_features.shape[1]
    L = m_w0.shape[0]
    inv_scale = 1.0 / 30.0

    B = 400
    if N % B != 0:
        for cand in (256, 200, 128, 100, 80, 50, 40, 25, 20, 16, 10, 8):
            if N % cand == 0:
                B = cand
                break
        else:
            B = N
    grid = (N // B,)

    mask2d = mask.reshape(N, 1)

    m_w0_s = m_w0 * _RS2
    m_b0_s = m_b0 * _RS2
    m_w1_s = m_w1 * 0.5
    m_b1_s = m_b1 * _RS2
    m_w2_s = m_w2 * (inv_scale / (2.0 * _RS2))
    m_b2_s = m_b2 * (K * inv_scale)
    d_w0_s = d_w0 * _RS2
    d_b0_s = d_b0 * _RS2
    d_w1_s = d_w1 / (2.0 * _RS2)
    a8 = (jnp.arange(8 * B)[None, :] // 8
          == jnp.arange(B)[:, None]).astype(jnp.float32)

    full = lambda a: pl.BlockSpec(a.shape, lambda i: (0,) * a.ndim)

    out = pl.pallas_call(
        functools.partial(_decoder_block, num_layers=L),
        grid=grid,
        in_specs=[
            pl.BlockSpec((B, C), lambda i: (i, 0)),
            pl.BlockSpec((B, K, E), lambda i: (i, 0, 0)),
            pl.BlockSpec((B, 1), lambda i: (i, 0)),
            full(a8),
            full(m_w0_s), full(m_b0_s), full(m_w1_s), full(m_b1_s),
            full(m_w2_s), full(m_b2_s), full(n1_w), full(n1_b),
            full(d_w0_s), full(d_b0_s), full(d_w1_s), full(d_b1),
            full(n2_w), full(n2_b),
        ],
        out_specs=pl.BlockSpec((B, C), lambda i: (i, 0)),
        out_shape=jax.ShapeDtypeStruct((N, C), node_features.dtype),
    )(node_features, edge_features, mask2d, a8,
      m_w0_s, m_b0_s, m_w1_s, m_b1_s,
      m_w2_s, m_b2_s, n1_w, n1_b, d_w0_s, d_b0_s, d_w1_s, d_b1, n2_w, n2_b)
    return out


# speedup vs baseline: 1.1484x; 1.1484x over previous
import functools

import jax
import jax.numpy as jnp
from jax.experimental import pallas as pl

_RS2 = 0.7071067811865476


def _g2(t):
    return t * (1.0 + jax.lax.erf(t))


def _ln(x, g, b, eps=1e-05):
    mu = jnp.mean(x, axis=-1, keepdims=True)
    d = x - mu
    var = jnp.mean(d * d, axis=-1, keepdims=True)
    return d * jax.lax.rsqrt(var + eps) * g + b


def _decoder_block(node_ref, edge_ref, mask_ref, mw0_ref, mb0_ref,
                   mw1_ref, mb1_ref, mw2_ref, mb2_ref, n1w_ref, n1b_ref,
                   dw0_ref, db0_ref, dw1_ref, db1_ref, n2w_ref, n2b_ref,
                   out_ref, *, num_layers):
    B, K, E = edge_ref.shape
    C = node_ref.shape[1]

    x = node_ref[...]
    e = edge_ref[...].reshape(B * K, E)
    msk = mask_ref[...]
    h = x
    te = e @ mw0_ref[0][:, 3 * C:].T
    for i in range(num_layers):
        w0 = mw0_ref[i]
        wh = w0[:, :C]
        wn = w0[:, C:2 * C]
        per_node = h @ wh.T + x @ wn.T + mb0_ref[i]
        t = (te.reshape(B, K, C) + per_node[:, None, :]).reshape(B * K, C)
        m = _g2(t)
        m = _g2(m @ mw1_ref[i].T + mb1_ref[i])
        s = jnp.sum(m.reshape(B, K, C), axis=1)
        if i + 1 < num_layers:
            te = e @ mw0_ref[i + 1][:, 3 * C:].T
        h = h + s @ mw2_ref[i].T + mb2_ref[i]
        h1 = _ln(h, n1w_ref[i], n1b_ref[i])
        dns = _g2(h1 @ dw0_ref[i].T + db0_ref[i])
        h = h1 + dns @ dw1_ref[i].T + db1_ref[i]
        h = msk * _ln(h, n2w_ref[i], n2b_ref[i])
    out_ref[...] = h


def kernel(node_features, edge_features, mask, m_w0, m_b0, m_w1, m_b1, m_w2,
           m_b2, n1_w, n1_b, d_w0, d_b0, d_w1, d_b1, n2_w, n2_b):
    N, K, E = edge_features.shape
    C = node_features.shape[1]
    L = m_w0.shape[0]
    inv_scale = 1.0 / 30.0

    B = 400
    if N % B != 0:
        for cand in (256, 200, 128, 100, 80, 50, 40, 25, 20, 16, 10, 8):
            if N % cand == 0:
                B = cand
                break
        else:
            B = N
    grid = (N // B,)

    mask2d = mask.reshape(N, 1)

    m_w0_s = m_w0 * _RS2
    m_b0_s = m_b0 * _RS2
    m_w1_s = m_w1 * 0.5
    m_b1_s = m_b1 * _RS2
    m_w2_s = m_w2 * (inv_scale / (2.0 * _RS2))
    m_b2_s = m_b2 * (K * inv_scale)
    d_w0_s = d_w0 * _RS2
    d_b0_s = d_b0 * _RS2
    d_w1_s = d_w1 / (2.0 * _RS2)

    full = lambda a: pl.BlockSpec(a.shape, lambda i: (0,) * a.ndim)

    out = pl.pallas_call(
        functools.partial(_decoder_block, num_layers=L),
        grid=grid,
        in_specs=[
            pl.BlockSpec((B, C), lambda i: (i, 0)),
            pl.BlockSpec((B, K, E), lambda i: (i, 0, 0)),
            pl.BlockSpec((B, 1), lambda i: (i, 0)),
            full(m_w0_s), full(m_b0_s), full(m_w1_s), full(m_b1_s),
            full(m_w2_s), full(m_b2_s), full(n1_w), full(n1_b),
            full(d_w0_s), full(d_b0_s), full(d_w1_s), full(d_b1),
            full(n2_w), full(n2_b),
        ],
        out_specs=pl.BlockSpec((B, C), lambda i: (i, 0)),
        out_shape=jax.ShapeDtypeStruct((N, C), node_features.dtype),
    )(node_features, edge_features, mask2d, m_w0_s, m_b0_s, m_w1_s, m_b1_s,
      m_w2_s, m_b2_s, n1_w, n1_b, d_w0_s, d_b0_s, d_w1_s, d_b1, n2_w, n2_b)
    return out


# speedup vs baseline: 1.2405x; 1.0801x over previous
import functools

import jax
import jax.numpy as jnp
from jax.experimental import pallas as pl

_RS2 = 0.7071067811865476


def _g2(t):
    return t * (1.0 + jax.lax.erf(t))


def _ln(x, g, b, eps=1e-05):
    mu = jnp.mean(x, axis=-1, keepdims=True)
    d = x - mu
    var = jnp.mean(d * d, axis=-1, keepdims=True)
    return d * jax.lax.rsqrt(var + eps) * g + b


def _decoder_block(node_ref, edge_ref, mask_ref, mw0_ref, mb0_ref,
                   mw1_ref, mb1_ref, mw2_ref, mb2_ref, n1w_ref, n1b_ref,
                   dw0_ref, db0_ref, dw1_ref, db1_ref, n2w_ref, n2b_ref,
                   out_ref, *, num_layers):
    B, K, E = edge_ref.shape
    C = node_ref.shape[1]

    x = node_ref[...]
    e = edge_ref[...].reshape(B * K, E)
    msk = mask_ref[...]
    h = x
    te = e @ mw0_ref[0][:, 3 * C:].T
    for i in range(num_layers):
        w0 = mw0_ref[i]
        wh = w0[:, :C]
        wn = w0[:, C:2 * C]
        per_node = h @ wh.T + x @ wn.T + mb0_ref[i]
        t = (te.reshape(B, K, C) + per_node[:, None, :]).reshape(B * K, C)
        m = _g2(t)
        m = _g2(m @ mw1_ref[i].T + mb1_ref[i])
        s = jnp.sum(m.reshape(B, K, C), axis=1)
        if i + 1 < num_layers:
            te = e @ mw0_ref[i + 1][:, 3 * C:].T
        h = h + s @ mw2_ref[i].T + mb2_ref[i]
        h1 = _ln(h, n1w_ref[i], n1b_ref[i])
        dns = _g2(h1 @ dw0_ref[i].T + db0_ref[i])
        h = h1 + dns @ dw1_ref[i].T + db1_ref[i]
        h = msk * _ln(h, n2w_ref[i], n2b_ref[i])
    out_ref[...] = h


def kernel(node_features, edge_features, mask, m_w0, m_b0, m_w1, m_b1, m_w2,
           m_b2, n1_w, n1_b, d_w0, d_b0, d_w1, d_b1, n2_w, n2_b):
    N, K, E = edge_features.shape
    C = node_features.shape[1]
    L = m_w0.shape[0]
    inv_scale = 1.0 / 30.0

    B = 800
    grid = ((N + B - 1) // B,)

    mask2d = mask.reshape(N, 1)

    m_w0_s = m_w0 * _RS2
    m_b0_s = m_b0 * _RS2
    m_w1_s = m_w1 * 0.5
    m_b1_s = m_b1 * _RS2
    m_w2_s = m_w2 * (inv_scale / (2.0 * _RS2))
    m_b2_s = m_b2 * (K * inv_scale)
    d_w0_s = d_w0 * _RS2
    d_b0_s = d_b0 * _RS2
    d_w1_s = d_w1 / (2.0 * _RS2)

    full = lambda a: pl.BlockSpec(a.shape, lambda i: (0,) * a.ndim)

    out = pl.pallas_call(
        functools.partial(_decoder_block, num_layers=L),
        grid=grid,
        in_specs=[
            pl.BlockSpec((B, C), lambda i: (i, 0)),
            pl.BlockSpec((B, K, E), lambda i: (i, 0, 0)),
            pl.BlockSpec((B, 1), lambda i: (i, 0)),
            full(m_w0_s), full(m_b0_s), full(m_w1_s), full(m_b1_s),
            full(m_w2_s), full(m_b2_s), full(n1_w), full(n1_b),
            full(d_w0_s), full(d_b0_s), full(d_w1_s), full(d_b1),
            full(n2_w), full(n2_b),
        ],
        out_specs=pl.BlockSpec((B, C), lambda i: (i, 0)),
        out_shape=jax.ShapeDtypeStruct((N, C), node_features.dtype),
    )(node_features, edge_features, mask2d, m_w0_s, m_b0_s, m_w1_s, m_b1_s,
      m_w2_s, m_b2_s, n1_w, n1_b, d_w0_s, d_b0_s, d_w1_s, d_b1, n2_w, n2_b)
    return out


# speedup vs baseline: 1.2455x; 1.0040x over previous
import functools

import jax
import jax.numpy as jnp
from jax.experimental import pallas as pl

_RS2 = 0.7071067811865476


def _g2(t):
    return t * (1.0 + jax.lax.erf(t))


def _ln(x, g, b, eps=1e-05):
    mu = jnp.mean(x, axis=-1, keepdims=True)
    d = x - mu
    var = jnp.mean(d * d, axis=-1, keepdims=True)
    return d * jax.lax.rsqrt(var + eps) * g + b


def _decoder_block(node_ref, edge_ref, mask_ref, mw0_ref, mb0_ref,
                   mw1_ref, mb1_ref, mw2_ref, mb2_ref, n1w_ref, n1b_ref,
                   dw0_ref, db0_ref, dw1_ref, db1_ref, n2w_ref, n2b_ref,
                   out_ref, *, num_layers):
    B, K, E = edge_ref.shape
    C = node_ref.shape[1]

    x = node_ref[...]
    e = edge_ref[...].reshape(B * K, E)
    msk = mask_ref[...]
    h = x
    te = e @ mw0_ref[0][:, 3 * C:].T
    for i in range(num_layers):
        w0 = mw0_ref[i]
        wh = w0[:, :C]
        wn = w0[:, C:2 * C]
        per_node = h @ wh.T + x @ wn.T + mb0_ref[i]
        t = (te.reshape(B, K, C) + per_node[:, None, :]).reshape(B * K, C)
        m = _g2(t)
        m = _g2(m @ mw1_ref[i].T + mb1_ref[i])
        s = jnp.sum(m.reshape(B, K, C), axis=1)
        if i + 1 < num_layers:
            te = e @ mw0_ref[i + 1][:, 3 * C:].T
        h = h + s @ mw2_ref[i].T + mb2_ref[i]
        h1 = _ln(h, n1w_ref[i], n1b_ref[i])
        dns = _g2(h1 @ dw0_ref[i].T + db0_ref[i])
        h = h1 + dns @ dw1_ref[i].T + db1_ref[i]
        h = msk * _ln(h, n2w_ref[i], n2b_ref[i])
    out_ref[...] = h


def kernel(node_features, edge_features, mask, m_w0, m_b0, m_w1, m_b1, m_w2,
           m_b2, n1_w, n1_b, d_w0, d_b0, d_w1, d_b1, n2_w, n2_b):
    N, K, E = edge_features.shape
    C = node_features.shape[1]
    L = m_w0.shape[0]
    inv_scale = 1.0 / 30.0

    B = 1000
    grid = ((N + B - 1) // B,)

    mask2d = mask.reshape(N, 1)

    m_w0_s = m_w0 * _RS2
    m_b0_s = m_b0 * _RS2
    m_w1_s = m_w1 * 0.5
    m_b1_s = m_b1 * _RS2
    m_w2_s = m_w2 * (inv_scale / (2.0 * _RS2))
    m_b2_s = m_b2 * (K * inv_scale)
    d_w0_s = d_w0 * _RS2
    d_b0_s = d_b0 * _RS2
    d_w1_s = d_w1 / (2.0 * _RS2)

    full = lambda a: pl.BlockSpec(a.shape, lambda i: (0,) * a.ndim)

    out = pl.pallas_call(
        functools.partial(_decoder_block, num_layers=L),
        grid=grid,
        in_specs=[
            pl.BlockSpec((B, C), lambda i: (i, 0)),
            pl.BlockSpec((B, K, E), lambda i: (i, 0, 0)),
            pl.BlockSpec((B, 1), lambda i: (i, 0)),
            full(m_w0_s), full(m_b0_s), full(m_w1_s), full(m_b1_s),
            full(m_w2_s), full(m_b2_s), full(n1_w), full(n1_b),
            full(d_w0_s), full(d_b0_s), full(d_w1_s), full(d_b1),
            full(n2_w), full(n2_b),
        ],
        out_specs=pl.BlockSpec((B, C), lambda i: (i, 0)),
        out_shape=jax.ShapeDtypeStruct((N, C), node_features.dtype),
    )(node_features, edge_features, mask2d, m_w0_s, m_b0_s, m_w1_s, m_b1_s,
      m_w2_s, m_b2_s, n1_w, n1_b, d_w0_s, d_b0_s, d_w1_s, d_b1, n2_w, n2_b)
    return out
